# baseline (speedup 1.0000x reference)
import jax
import jax.numpy as jnp
from jax.experimental import pallas as pl
from jax.experimental.pallas import tpu as pltpu

_DELTA = 0.1
_EPS = 1e-12
_BR = 2048
_BC = 2048
_SENTINEL = 1e9


def _count_chunk(gcols_ref, colcnt_ref, g_r, off, acc, bank=0,
                 skip_first_col=False):
    for s in range(_BC // 128):
        gc = gcols_ref[:, pl.ds(off + s * 128, 128)]
        m = (jnp.abs(g_r - gc) <= _DELTA).astype(jnp.float32)
        acc = acc + m
        if not (skip_first_col and s == 0):
            cs = m[0:8]
            for k in range(1, 16):
                cs = cs + m[k * 8:(k + 1) * 8]
            colcnt_ref[bank:bank + 8, pl.ds(off + s * 128, 128)] += cs
    return acc


def _ghm_body(x_rows_ref, y_rows_ref, x_cols_ref, y_cols_ref, pw_ref,
              wsum_ref, psum_ref, gcols_ref, colcnt_ref, rowcnt_ref):
    i = pl.program_id(0)
    n = x_cols_ref.shape[1]
    ng = pl.num_programs(0)
    nslabs = n // 128

    @pl.when(i == 0)
    def _init():
        xc = x_cols_ref[...]
        yc = y_cols_ref[...]
        gcols_ref[:, :n] = jnp.abs(jax.nn.sigmoid(xc) - yc)
        gcols_ref[:, n:] = jnp.full((1, _BC), _SENTINEL, jnp.float32)
        colcnt_ref[...] = jnp.zeros_like(colcnt_ref)

    r0 = i * _BR

    for rh in range(_BR // 128):
        x_rh = x_rows_ref[0, rh * 128:(rh + 1) * 128]
        y_rh = y_rows_ref[0, rh * 128:(rh + 1) * 128]
        g_rh = jnp.abs(jax.nn.sigmoid(x_rh) - y_rh)

        gr = r0 // 128 + rh
        base = pl.multiple_of(gr * 128, 128)

        bank = 8 * (rh % 2)
        acc = _count_chunk(gcols_ref, colcnt_ref, g_rh, base,
                           jnp.zeros((128, 128), jnp.float32), bank,
                           skip_first_col=True)

        def chunk(c, a, g_rh=g_rh, base=base, bank=bank):
            off = pl.multiple_of(base + c * _BC, 128)
            return _count_chunk(gcols_ref, colcnt_ref, g_rh, off, a, bank)

        cw = _BC // 128
        ntrips = 1 + (nslabs - gr - cw + cw - 1) // cw
        acc = jax.lax.fori_loop(1, ntrips, chunk, acc)

        cnt = jnp.sum(acc, axis=1, keepdims=True)
        rowcnt_ref[:, pl.ds(r0 + rh * 128, 128)] = cnt.reshape(1, 128)

    @pl.when(i == ng - 1)
    def _finalize():
        colsum = jnp.sum(colcnt_ref[:, :n], axis=0, keepdims=True)
        gd = (rowcnt_ref[...] + colsum) / _DELTA
        beta = n / (gd + _EPS)
        xc = x_cols_ref[...]
        yc = y_cols_ref[...]
        pw = pw_ref[0, 0]
        pe = (pw * yc * jax.nn.softplus(-xc)
              + (1.0 - yc) * jax.nn.softplus(xc))
        wsum_ref[0, 0] = jnp.sum(beta * pe)
        psum_ref[0, 0] = jnp.sum(pe)


def kernel(logits, targets, pos_weight):
    x = logits.reshape(-1).astype(jnp.float32)
    y = targets.reshape(-1).astype(jnp.float32)
    n = x.shape[0]
    g = n // _BR

    x_rows = x.reshape(g, _BR, 1)
    y_rows = y.reshape(g, _BR, 1)
    x_cols = x.reshape(1, n)
    y_cols = y.reshape(1, n)
    pw = jnp.asarray(pos_weight, jnp.float32).reshape(1, 1)

    wsum, psum = pl.pallas_call(
        _ghm_body,
        grid=(g,),
        in_specs=[
            pl.BlockSpec((1, _BR, 1), lambda i: (i, 0, 0)),
            pl.BlockSpec((1, _BR, 1), lambda i: (i, 0, 0)),
            pl.BlockSpec((1, n), lambda i: (0, 0)),
            pl.BlockSpec((1, n), lambda i: (0, 0)),
            pl.BlockSpec(memory_space=pltpu.SMEM),
        ],
        out_specs=[
            pl.BlockSpec((1, 1), lambda i: (0, 0), memory_space=pltpu.SMEM),
            pl.BlockSpec((1, 1), lambda i: (0, 0), memory_space=pltpu.SMEM),
        ],
        out_shape=[
            jax.ShapeDtypeStruct((1, 1), jnp.float32),
            jax.ShapeDtypeStruct((1, 1), jnp.float32),
        ],
        scratch_shapes=[
            pltpu.VMEM((1, n + _BC), jnp.float32),
            pltpu.VMEM((16, n + _BC), jnp.float32),
            pltpu.VMEM((1, n), jnp.float32),
        ],
        compiler_params=pltpu.CompilerParams(
            dimension_semantics=("arbitrary",),
        ),
        name="ghm_bce",
    )(x_rows, y_rows, x_cols, y_cols, pw)

    inv_n = jnp.float32(1.0 / n)
    return wsum[0, 0] * inv_n, psum[0, 0] * inv_n


# speedup vs baseline: 1.0022x; 1.0022x over previous
import jax
import jax.numpy as jnp
from jax.experimental import pallas as pl
from jax.experimental.pallas import tpu as pltpu

_DELTA = 0.1
_EPS = 1e-12
_BR = 2048
_BC = 2048
_SENTINEL = 1e9


def _count_chunk(gcols_ref, colcnt_ref, g_r, off, acc, skip_first_col=False):
    for s in range(_BC // 128):
        gc = gcols_ref[:, pl.ds(off + s * 128, 128)]
        m = (jnp.abs(g_r - gc) <= _DELTA).astype(jnp.float32)
        acc = acc + m
        if not (skip_first_col and s == 0):
            cs = m[0:8]
            for k in range(1, 16):
                cs = cs + m[k * 8:(k + 1) * 8]
            colcnt_ref[:, pl.ds(off + s * 128, 128)] += cs
    return acc


def _ghm_body(x_rows_ref, y_rows_ref, x_cols_ref, y_cols_ref, pw_ref,
              wsum_ref, psum_ref, gcols_ref, colcnt_ref, rowcnt_ref):
    i = pl.program_id(0)
    n = x_cols_ref.shape[1]
    ng = pl.num_programs(0)
    nslabs = n // 128

    @pl.when(i == 0)
    def _init():
        xc = x_cols_ref[...]
        yc = y_cols_ref[...]
        gcols_ref[:, :n] = jnp.abs(jax.nn.sigmoid(xc) - yc)
        gcols_ref[:, n:] = jnp.full((1, _BC), _SENTINEL, jnp.float32)
        colcnt_ref[...] = jnp.zeros_like(colcnt_ref)

    r0 = i * _BR

    for rh in range(_BR // 128):
        x_rh = x_rows_ref[0, rh * 128:(rh + 1) * 128]
        y_rh = y_rows_ref[0, rh * 128:(rh + 1) * 128]
        g_rh = jnp.abs(jax.nn.sigmoid(x_rh) - y_rh)

        gr = r0 // 128 + rh
        base = pl.multiple_of(gr * 128, 128)

        acc = _count_chunk(gcols_ref, colcnt_ref, g_rh, base,
                           jnp.zeros((128, 128), jnp.float32),
                           skip_first_col=True)

        def chunk(c, a, g_rh=g_rh, base=base):
            off = pl.multiple_of(base + c * _BC, 128)
            return _count_chunk(gcols_ref, colcnt_ref, g_rh, off, a)

        cw = _BC // 128
        ntrips = 1 + (nslabs - gr - cw + cw - 1) // cw
        acc = jax.lax.fori_loop(1, ntrips, chunk, acc)

        cnt = jnp.sum(acc, axis=1, keepdims=True)
        rowcnt_ref[:, pl.ds(r0 + rh * 128, 128)] = cnt.reshape(1, 128)

    @pl.when(i == ng - 1)
    def _finalize():
        colsum = jnp.sum(colcnt_ref[:, :n], axis=0, keepdims=True)
        gd = (rowcnt_ref[...] + colsum) / _DELTA
        beta = n / (gd + _EPS)
        xc = x_cols_ref[...]
        yc = y_cols_ref[...]
        pw = pw_ref[0, 0]
        pe = (pw * yc * jax.nn.softplus(-xc)
              + (1.0 - yc) * jax.nn.softplus(xc))
        wsum_ref[0, 0] = jnp.sum(beta * pe)
        psum_ref[0, 0] = jnp.sum(pe)


def kernel(logits, targets, pos_weight):
    x = logits.reshape(-1).astype(jnp.float32)
    y = targets.reshape(-1).astype(jnp.float32)
    n = x.shape[0]
    g = n // _BR

    x_rows = x.reshape(g, _BR, 1)
    y_rows = y.reshape(g, _BR, 1)
    x_cols = x.reshape(1, n)
    y_cols = y.reshape(1, n)
    pw = jnp.asarray(pos_weight, jnp.float32).reshape(1, 1)

    wsum, psum = pl.pallas_call(
        _ghm_body,
        grid=(g,),
        in_specs=[
            pl.BlockSpec((1, _BR, 1), lambda i: (i, 0, 0)),
            pl.BlockSpec((1, _BR, 1), lambda i: (i, 0, 0)),
            pl.BlockSpec((1, n), lambda i: (0, 0)),
            pl.BlockSpec((1, n), lambda i: (0, 0)),
            pl.BlockSpec(memory_space=pltpu.SMEM),
        ],
        out_specs=[
            pl.BlockSpec((1, 1), lambda i: (0, 0), memory_space=pltpu.SMEM),
            pl.BlockSpec((1, 1), lambda i: (0, 0), memory_space=pltpu.SMEM),
        ],
        out_shape=[
            jax.ShapeDtypeStruct((1, 1), jnp.float32),
            jax.ShapeDtypeStruct((1, 1), jnp.float32),
        ],
        scratch_shapes=[
            pltpu.VMEM((1, n + _BC), jnp.float32),
            pltpu.VMEM((8, n + _BC), jnp.float32),
            pltpu.VMEM((1, n), jnp.float32),
        ],
        compiler_params=pltpu.CompilerParams(
            dimension_semantics=("arbitrary",),
        ),
        name="ghm_bce",
    )(x_rows, y_rows, x_cols, y_cols, pw)

    inv_n = jnp.float32(1.0 / n)
    return wsum[0, 0] * inv_n, psum[0, 0] * inv_n
